# baseline (speedup 1.0000x reference)
import jax
import jax.numpy as jnp
from jax import lax
from jax.experimental import pallas as pl
from jax.experimental.pallas import tpu as pltpu
from jax.experimental.pallas import tpu_sc as plsc

N = 10000
NPAD = 10240
D = 256
DH = 128
E = 160000
BN_EPS = 1e-3
RPT = NPAD // 16
PK = NPAD // 128

CH = 125
NCH_R = E // (16 * CH)
NH_R = NCH_R // 2
NBUF = 4

NCH_S = E // (32 * CH)

_MESH = plsc.VectorSubcoreMesh(core_axis_name="c", subcore_axis_name="s")
_NOTILE = pltpu.CompilerParams(use_tc_tiling_on_sc=False,
                               needs_layout_passes=False)


def _pack_col0(acc, colbuf, pack, out_hbm, cid, sid):
    pltpu.sync_copy(acc.at[pl.ds(sid * RPT, RPT)], colbuf)
    row16 = lax.broadcasted_iota(jnp.int32, (16,), 0)
    zero16 = jnp.zeros((16,), jnp.int32)
    for g in range(RPT // 16):
        vals = plsc.load_gather(colbuf, [g * 16 + row16, zero16])
        pack[g // 8, pl.ds((g % 8) * 16, 16)] = vals
    pltpu.sync_copy(pack, out_hbm.at[cid, pl.ds(sid * (RPT // 128), RPT // 128)])


def _sc_rowsum_body(hs_hbm, src_hbm, dst_hbm, iidx_hbm, out_hbm,
                    src_idx, dst_idx, rows0, rows1, rows2, rows3, acc,
                    iidx_v, ibuf,
                    g0, g1, g2, g3, s0, s1, s2, s3):
    rows = (rows0, rows1, rows2, rows3)
    gsem = (g0, g1, g2, g3)
    ssem = (s0, s1, s2, s3)
    cid = lax.axis_index("c")
    sid = lax.axis_index("s")
    pltpu.sync_copy(iidx_hbm.at[cid, sid], iidx_v)
    for k in range(RPT // 128):
        pltpu.async_copy(hs_hbm.at[iidx_v.at[k]], ibuf, g0).wait()
        pltpu.sync_copy(ibuf, acc.at[pl.ds(sid * RPT + k * 128, 128)])
    plsc.subcore_barrier()
    for h in range(2):
        pltpu.sync_copy(src_hbm.at[cid, sid, pl.ds(h * NH_R, NH_R)], src_idx)
        pltpu.sync_copy(dst_hbm.at[sid, pl.ds(h * NH_R, NH_R)], dst_idx)
        for b in range(NBUF):
            pltpu.async_copy(hs_hbm.at[src_idx.at[b]], rows[b], gsem[b])

        def step(i, carry):
            for b in range(NBUF):
                j = NBUF * i + b
                pltpu.make_async_copy(hs_hbm.at[src_idx.at[j]], rows[b],
                                      gsem[b]).wait()
                pltpu.async_copy(rows[b], acc.at[dst_idx.at[j]], ssem[b],
                                 add=True)
            for b in range(NBUF):
                j = NBUF * i + b

                @pl.when(j + NBUF < NH_R)
                def _():
                    pltpu.make_async_copy(rows[b], acc.at[dst_idx.at[j]],
                                          ssem[b]).wait()
                    pltpu.async_copy(hs_hbm.at[src_idx.at[j + NBUF]], rows[b],
                                     gsem[b])

            return carry

        lax.fori_loop(0, NH_R // NBUF, step, 0)
        for b in range(NBUF):
            pltpu.make_async_copy(rows[b], acc.at[dst_idx.at[0]],
                                  ssem[b]).wait()
    plsc.subcore_barrier()
    pltpu.sync_copy(acc.at[pl.ds(sid * RPT, RPT)],
                    out_hbm.at[pl.ds(sid * RPT, RPT), cid])


_sc_rowsum = pl.kernel(
    _sc_rowsum_body,
    out_type=jax.ShapeDtypeStruct((NPAD, 2, DH), jnp.bfloat16),
    mesh=_MESH,
    compiler_params=_NOTILE,
    scratch_types=[
        pltpu.VMEM((NH_R, CH), jnp.int32),
        pltpu.VMEM((NH_R, CH), jnp.int32),
        pltpu.VMEM((CH, DH), jnp.bfloat16),
        pltpu.VMEM((CH, DH), jnp.bfloat16),
        pltpu.VMEM((CH, DH), jnp.bfloat16),
        pltpu.VMEM((CH, DH), jnp.bfloat16),
        pltpu.VMEM_SHARED((NPAD, DH), jnp.bfloat16),
        pltpu.VMEM((RPT // 128, 128), jnp.int32),
        pltpu.VMEM((128, DH), jnp.bfloat16),
    ] + [pltpu.SemaphoreType.DMA] * 8,
)


def _sc_segsum16_body(tab_hbm, src_hbm, dst_hbm, out_hbm,
                      src_idx, dst_idx, rows0, rows1, colbuf, pack, acc,
                      sem0, sem1):
    cid = lax.axis_index("c")
    sid = lax.axis_index("s")
    wid = cid * 16 + sid
    pltpu.sync_copy(src_hbm.at[wid], src_idx)
    pltpu.sync_copy(dst_hbm.at[wid], dst_idx)
    pltpu.sync_copy(tab_hbm.at[pl.ds(sid * RPT, RPT)],
                    acc.at[pl.ds(sid * RPT, RPT)])
    plsc.subcore_barrier()
    pltpu.async_copy(tab_hbm.at[src_idx.at[0]], rows0, sem0)
    pltpu.async_copy(tab_hbm.at[src_idx.at[1]], rows1, sem1)

    def step(i, carry):
        j0 = 2 * i
        j1 = 2 * i + 1
        pltpu.make_async_copy(tab_hbm.at[src_idx.at[j0]], rows0, sem0).wait()
        pltpu.sync_copy(rows0, acc.at[dst_idx.at[j0]], add=True)

        @pl.when(j0 + 2 < NCH_S)
        def _():
            pltpu.async_copy(tab_hbm.at[src_idx.at[j0 + 2]], rows0, sem0)

        pltpu.make_async_copy(tab_hbm.at[src_idx.at[j1]], rows1, sem1).wait()
        pltpu.sync_copy(rows1, acc.at[dst_idx.at[j1]], add=True)

        @pl.when(j1 + 2 < NCH_S)
        def _():
            pltpu.async_copy(tab_hbm.at[src_idx.at[j1 + 2]], rows1, sem1)

        return carry

    lax.fori_loop(0, NCH_S // 2, step, 0)
    plsc.subcore_barrier()
    _pack_col0(acc, colbuf, pack, out_hbm, cid, sid)


_sc_segsum16 = pl.kernel(
    _sc_segsum16_body,
    out_type=jax.ShapeDtypeStruct((2, PK, 128), jnp.float32),
    mesh=_MESH,
    compiler_params=_NOTILE,
    scratch_types=[
        pltpu.VMEM((NCH_S, CH), jnp.int32),
        pltpu.VMEM((NCH_S, CH), jnp.int32),
        pltpu.VMEM((CH, 16), jnp.float32),
        pltpu.VMEM((CH, 16), jnp.float32),
        pltpu.VMEM((RPT, 16), jnp.float32),
        pltpu.VMEM((RPT // 128, 128), jnp.float32),
        pltpu.VMEM_SHARED((NPAD, 16), jnp.float32),
        pltpu.SemaphoreType.DMA,
        pltpu.SemaphoreType.DMA,
    ],
)


def _sc_degree_body(dst_hbm, ones_hbm, out_hbm, dst_idx, ones_v, colbuf,
                    pack, acc):
    cid = lax.axis_index("c")
    sid = lax.axis_index("s")
    wid = cid * 16 + sid
    pltpu.sync_copy(dst_hbm.at[wid], dst_idx)
    pltpu.sync_copy(ones_hbm, ones_v)
    for k in range(RPT // CH + 1):
        base = sid * RPT + k * CH
        size = min(CH, RPT - k * CH)
        if size > 0:
            pltpu.sync_copy(ones_v.at[pl.ds(0, size)],
                            acc.at[pl.ds(base, size)])
    plsc.subcore_barrier()

    def step(j, carry):
        pltpu.sync_copy(ones_v, acc.at[dst_idx.at[j]], add=True)
        return carry

    lax.fori_loop(0, NCH_S, step, 0)
    plsc.subcore_barrier()
    _pack_col0(acc, colbuf, pack, out_hbm, cid, sid)


_sc_degree = pl.kernel(
    _sc_degree_body,
    out_type=jax.ShapeDtypeStruct((2, PK, 128), jnp.float32),
    mesh=_MESH,
    compiler_params=_NOTILE,
    scratch_types=[
        pltpu.VMEM((NCH_S, CH), jnp.int32),
        pltpu.VMEM((CH, 16), jnp.float32),
        pltpu.VMEM((RPT, 16), jnp.float32),
        pltpu.VMEM((RPT // 128, 128), jnp.float32),
        pltpu.VMEM_SHARED((NPAD, 16), jnp.float32),
    ],
)


def _tc_prep_body(x_ref, w1_ref, dc_ref, hs_ref):
    dinv = lax.rsqrt(dc_ref[0:N, :])
    h = jnp.dot(x_ref[...], w1_ref[...], preferred_element_type=jnp.float32)
    hs_ref[0:N, :] = h * dinv
    hs_ref[N:NPAD, :] = jnp.zeros((NPAD - N, D), jnp.float32)


def _tc_mid_body(agg_ref, dc_ref, b1_ref, g1_ref, be1_ref, w2_ref, out_ref):
    dinv = lax.rsqrt(dc_ref[...])
    aggl = agg_ref[:, 0:DH]
    aggr = agg_ref[:, DH:D]
    zl = jnp.maximum(aggl * dinv + b1_ref[:, 0:DH], 0.0)
    zr = jnp.maximum(aggr * dinv + b1_ref[:, DH:D], 0.0)
    ml = jnp.sum(zl[0:N, :], axis=0, keepdims=True) * (1.0 / N)
    mr = jnp.sum(zr[0:N, :], axis=0, keepdims=True) * (1.0 / N)
    ql = jnp.sum(zl[0:N, :] * zl[0:N, :], axis=0, keepdims=True) * (1.0 / N)
    qr = jnp.sum(zr[0:N, :] * zr[0:N, :], axis=0, keepdims=True) * (1.0 / N)
    il = lax.rsqrt(ql - ml * ml + BN_EPS)
    ir = lax.rsqrt(qr - mr * mr + BN_EPS)
    hl = jnp.maximum((zl - ml) * il * g1_ref[:, 0:DH] + be1_ref[:, 0:DH], 0.0)
    hr = jnp.maximum((zr - mr) * ir * g1_ref[:, DH:D] + be1_ref[:, DH:D], 0.0)
    v = (jnp.sum(hl * w2_ref[:, 0:DH], axis=1, keepdims=True)
         + jnp.sum(hr * w2_ref[:, DH:D], axis=1, keepdims=True))
    out_ref[...] = jnp.broadcast_to(v * dinv, (NPAD, 16))


def _tc_final_body(p_ref, dc_ref, vs_ref, b2_ref, g2_ref, be2_ref, out_ref):
    a2 = p_ref[0:N, :] - vs_ref[0:N, 0:1]
    o2 = a2 * lax.rsqrt(dc_ref[0:N, :]) + b2_ref[...]
    m = jnp.sum(o2, axis=0, keepdims=True) * (1.0 / N)
    q = jnp.sum(o2 * o2, axis=0, keepdims=True) * (1.0 / N)
    z = (o2 - m) * lax.rsqrt(q - m * m + BN_EPS) * g2_ref[...] + be2_ref[...]
    out_ref[...] = jax.nn.sigmoid(jnp.maximum(z, 0.0))


def kernel(x, pos_edge_index, edge_attr, W1, b1, bn1_gamma, bn1_beta,
           W2, b2, bn2_gamma, bn2_beta):
    f32 = jnp.float32
    src = pos_edge_index[0].astype(jnp.int32)
    dst = pos_edge_index[1].astype(jnp.int32)
    dst3 = dst.reshape(16, NCH_R, CH)
    src_h = src.reshape(32, NCH_S, CH)
    dst_h = dst.reshape(32, NCH_S, CH)
    ones16 = jnp.ones((CH, 16), f32)

    degparts = _sc_degree(dst_h, ones16)
    degcol = (degparts[0] + degparts[1] - 1.0).reshape(NPAD, 1)

    hsf = pl.pallas_call(
        _tc_prep_body,
        out_shape=jax.ShapeDtypeStruct((NPAD, D), f32),
    )(x, W1, degcol)
    hsb = hsf.astype(jnp.bfloat16)
    hs2 = hsb.reshape(2 * NPAD, DH)
    srcx = (2 * src[None, :] + jnp.arange(2, dtype=jnp.int32)[:, None])
    srcx = srcx.reshape(2, 16, NCH_R, CH)
    iidx = (2 * jnp.arange(NPAD, dtype=jnp.int32)[None, :]
            + jnp.arange(2, dtype=jnp.int32)[:, None]).reshape(2, 16, RPT // 128, 128)

    agg2 = _sc_rowsum(hs2, srcx, dst3, iidx)
    aggf = agg2.reshape(NPAD, D).astype(f32)

    vs2d = pl.pallas_call(
        _tc_mid_body,
        out_shape=jax.ShapeDtypeStruct((NPAD, 16), f32),
    )(aggf, degcol, b1.reshape(1, D), bn1_gamma.reshape(1, D),
      bn1_beta.reshape(1, D), W2.reshape(1, D))

    parts2 = _sc_segsum16(vs2d, src_h, dst_h)
    psum = (parts2[0] + parts2[1]).reshape(NPAD, 1)

    out = pl.pallas_call(
        _tc_final_body,
        out_shape=jax.ShapeDtypeStruct((N, 1), f32),
    )(psum, degcol, vs2d, b2.reshape(1, 1),
      bn2_gamma.reshape(1, 1), bn2_beta.reshape(1, 1))
    return out


# speedup vs baseline: 1.0805x; 1.0805x over previous
import jax
import jax.numpy as jnp
from jax import lax
from jax.experimental import pallas as pl
from jax.experimental.pallas import tpu as pltpu
from jax.experimental.pallas import tpu_sc as plsc

N = 10000
NPAD = 10240
D = 256
DH = 128
E = 160000
BN_EPS = 1e-3
RPT = NPAD // 16
PK = NPAD // 128

CH = 125
NCH_R = E // (16 * CH)
NH_R = NCH_R // 2
NBUF = 5

NCH_S = E // (32 * CH)

_MESH = plsc.VectorSubcoreMesh(core_axis_name="c", subcore_axis_name="s")
_NOTILE = pltpu.CompilerParams(use_tc_tiling_on_sc=False,
                               needs_layout_passes=False)


def _pack_col0(acc, colbuf, pack, out_hbm, cid, sid):
    pltpu.sync_copy(acc.at[pl.ds(sid * RPT, RPT)], colbuf)
    row16 = lax.broadcasted_iota(jnp.int32, (16,), 0)
    zero16 = jnp.zeros((16,), jnp.int32)
    for g in range(RPT // 16):
        vals = plsc.load_gather(colbuf, [g * 16 + row16, zero16])
        pack[g // 8, pl.ds((g % 8) * 16, 16)] = vals
    pltpu.sync_copy(pack, out_hbm.at[cid, pl.ds(sid * (RPT // 128), RPT // 128)])


def _sc_rowsum_body(hs_hbm, src_hbm, dst_hbm, out_hbm,
                    src_idx, dst_idx, rows0, rows1, rows2, rows3, rows4,
                    acc, g0, g1, g2, g3, g4, s0, s1, s2, s3, s4):
    rows = (rows0, rows1, rows2, rows3, rows4)
    gsem = (g0, g1, g2, g3, g4)
    ssem = (s0, s1, s2, s3, s4)
    cid = lax.axis_index("c")
    sid = lax.axis_index("s")
    tab = hs_hbm.at[cid]
    pltpu.sync_copy(tab.at[pl.ds(sid * RPT, RPT)],
                    acc.at[pl.ds(sid * RPT, RPT)])
    plsc.subcore_barrier()
    for h in range(2):
        pltpu.sync_copy(src_hbm.at[sid, pl.ds(h * NH_R, NH_R)], src_idx)
        pltpu.sync_copy(dst_hbm.at[sid, pl.ds(h * NH_R, NH_R)], dst_idx)
        for b in range(NBUF):
            pltpu.async_copy(tab.at[src_idx.at[b]], rows[b], gsem[b])

        def step(i, carry):
            for b in range(NBUF):
                j = NBUF * i + b
                pltpu.make_async_copy(tab.at[src_idx.at[j]], rows[b],
                                      gsem[b]).wait()
                pltpu.async_copy(rows[b], acc.at[dst_idx.at[j]], ssem[b],
                                 add=True)
            for b in range(NBUF):
                j = NBUF * i + b

                @pl.when(j + NBUF < NH_R)
                def _():
                    pltpu.make_async_copy(rows[b], acc.at[dst_idx.at[j]],
                                          ssem[b]).wait()
                    pltpu.async_copy(tab.at[src_idx.at[j + NBUF]], rows[b],
                                     gsem[b])

            return carry

        lax.fori_loop(0, NH_R // NBUF, step, 0)
        for b in range(NBUF):
            pltpu.make_async_copy(rows[b], acc.at[dst_idx.at[0]],
                                  ssem[b]).wait()
    plsc.subcore_barrier()
    pltpu.sync_copy(acc.at[pl.ds(sid * RPT, RPT)],
                    out_hbm.at[cid, pl.ds(sid * RPT, RPT)])


_sc_rowsum = pl.kernel(
    _sc_rowsum_body,
    out_type=jax.ShapeDtypeStruct((2, NPAD, DH), jnp.bfloat16),
    mesh=_MESH,
    compiler_params=_NOTILE,
    scratch_types=[
        pltpu.VMEM((NH_R, CH), jnp.int32),
        pltpu.VMEM((NH_R, CH), jnp.int32),
        pltpu.VMEM((CH, DH), jnp.bfloat16),
        pltpu.VMEM((CH, DH), jnp.bfloat16),
        pltpu.VMEM((CH, DH), jnp.bfloat16),
        pltpu.VMEM((CH, DH), jnp.bfloat16),
        pltpu.VMEM((CH, DH), jnp.bfloat16),
        pltpu.VMEM_SHARED((NPAD, DH), jnp.bfloat16),
    ] + [pltpu.SemaphoreType.DMA] * 10,
)


def _sc_segsum16_body(tab_hbm, src_hbm, dst_hbm, out_hbm,
                      src_idx, dst_idx, rows0, rows1, colbuf, pack, acc,
                      sem0, sem1):
    cid = lax.axis_index("c")
    sid = lax.axis_index("s")
    wid = cid * 16 + sid
    pltpu.sync_copy(src_hbm.at[wid], src_idx)
    pltpu.sync_copy(dst_hbm.at[wid], dst_idx)
    pltpu.sync_copy(tab_hbm.at[pl.ds(sid * RPT, RPT)],
                    acc.at[pl.ds(sid * RPT, RPT)])
    plsc.subcore_barrier()
    pltpu.async_copy(tab_hbm.at[src_idx.at[0]], rows0, sem0)
    pltpu.async_copy(tab_hbm.at[src_idx.at[1]], rows1, sem1)

    def step(i, carry):
        j0 = 2 * i
        j1 = 2 * i + 1
        pltpu.make_async_copy(tab_hbm.at[src_idx.at[j0]], rows0, sem0).wait()
        pltpu.sync_copy(rows0, acc.at[dst_idx.at[j0]], add=True)

        @pl.when(j0 + 2 < NCH_S)
        def _():
            pltpu.async_copy(tab_hbm.at[src_idx.at[j0 + 2]], rows0, sem0)

        pltpu.make_async_copy(tab_hbm.at[src_idx.at[j1]], rows1, sem1).wait()
        pltpu.sync_copy(rows1, acc.at[dst_idx.at[j1]], add=True)

        @pl.when(j1 + 2 < NCH_S)
        def _():
            pltpu.async_copy(tab_hbm.at[src_idx.at[j1 + 2]], rows1, sem1)

        return carry

    lax.fori_loop(0, NCH_S // 2, step, 0)
    plsc.subcore_barrier()
    _pack_col0(acc, colbuf, pack, out_hbm, cid, sid)


_sc_segsum16 = pl.kernel(
    _sc_segsum16_body,
    out_type=jax.ShapeDtypeStruct((2, PK, 128), jnp.float32),
    mesh=_MESH,
    compiler_params=_NOTILE,
    scratch_types=[
        pltpu.VMEM((NCH_S, CH), jnp.int32),
        pltpu.VMEM((NCH_S, CH), jnp.int32),
        pltpu.VMEM((CH, 16), jnp.float32),
        pltpu.VMEM((CH, 16), jnp.float32),
        pltpu.VMEM((RPT, 16), jnp.float32),
        pltpu.VMEM((RPT // 128, 128), jnp.float32),
        pltpu.VMEM_SHARED((NPAD, 16), jnp.float32),
        pltpu.SemaphoreType.DMA,
        pltpu.SemaphoreType.DMA,
    ],
)


def _sc_degree_body(dst_hbm, ones_hbm, out_hbm, dst_idx, ones_v, colbuf,
                    pack, acc):
    cid = lax.axis_index("c")
    sid = lax.axis_index("s")
    wid = cid * 16 + sid
    pltpu.sync_copy(dst_hbm.at[wid], dst_idx)
    pltpu.sync_copy(ones_hbm, ones_v)
    for k in range(RPT // CH + 1):
        base = sid * RPT + k * CH
        size = min(CH, RPT - k * CH)
        if size > 0:
            pltpu.sync_copy(ones_v.at[pl.ds(0, size)],
                            acc.at[pl.ds(base, size)])
    plsc.subcore_barrier()

    def step(j, carry):
        pltpu.sync_copy(ones_v, acc.at[dst_idx.at[j]], add=True)
        return carry

    lax.fori_loop(0, NCH_S, step, 0)
    plsc.subcore_barrier()
    _pack_col0(acc, colbuf, pack, out_hbm, cid, sid)


_sc_degree = pl.kernel(
    _sc_degree_body,
    out_type=jax.ShapeDtypeStruct((2, PK, 128), jnp.float32),
    mesh=_MESH,
    compiler_params=_NOTILE,
    scratch_types=[
        pltpu.VMEM((NCH_S, CH), jnp.int32),
        pltpu.VMEM((CH, 16), jnp.float32),
        pltpu.VMEM((RPT, 16), jnp.float32),
        pltpu.VMEM((RPT // 128, 128), jnp.float32),
        pltpu.VMEM_SHARED((NPAD, 16), jnp.float32),
    ],
)


def _tc_prep_body(x_ref, w1_ref, dc_ref, hs2_ref):
    dinv = lax.rsqrt(dc_ref[0:N, :])
    h = jnp.dot(x_ref[...], w1_ref[...], preferred_element_type=jnp.float32)
    hs = (h * dinv).astype(jnp.bfloat16)
    hs2_ref[0, 0:N, :] = hs[:, 0:DH]
    hs2_ref[1, 0:N, :] = hs[:, DH:D]
    hs2_ref[0, N:NPAD, :] = jnp.zeros((NPAD - N, DH), jnp.bfloat16)
    hs2_ref[1, N:NPAD, :] = jnp.zeros((NPAD - N, DH), jnp.bfloat16)


def _tc_mid_body(agg_ref, dc_ref, b1_ref, g1_ref, be1_ref, w2_ref, out_ref):
    dinv = lax.rsqrt(dc_ref[...])
    aggl = agg_ref[0].astype(jnp.float32)
    aggr = agg_ref[1].astype(jnp.float32)
    zl = jnp.maximum(aggl * dinv + b1_ref[:, 0:DH], 0.0)
    zr = jnp.maximum(aggr * dinv + b1_ref[:, DH:D], 0.0)
    ml = jnp.sum(zl[0:N, :], axis=0, keepdims=True) * (1.0 / N)
    mr = jnp.sum(zr[0:N, :], axis=0, keepdims=True) * (1.0 / N)
    ql = jnp.sum(zl[0:N, :] * zl[0:N, :], axis=0, keepdims=True) * (1.0 / N)
    qr = jnp.sum(zr[0:N, :] * zr[0:N, :], axis=0, keepdims=True) * (1.0 / N)
    il = lax.rsqrt(ql - ml * ml + BN_EPS)
    ir = lax.rsqrt(qr - mr * mr + BN_EPS)
    hl = jnp.maximum((zl - ml) * il * g1_ref[:, 0:DH] + be1_ref[:, 0:DH], 0.0)
    hr = jnp.maximum((zr - mr) * ir * g1_ref[:, DH:D] + be1_ref[:, DH:D], 0.0)
    v = (jnp.sum(hl * w2_ref[:, 0:DH], axis=1, keepdims=True)
         + jnp.sum(hr * w2_ref[:, DH:D], axis=1, keepdims=True))
    out_ref[...] = v * dinv


def _tc_final_body(p_ref, dc_ref, vs_ref, b2_ref, g2_ref, be2_ref, out_ref):
    a2 = p_ref[0:N, :] - vs_ref[0:N, :]
    o2 = a2 * lax.rsqrt(dc_ref[0:N, :]) + b2_ref[...]
    m = jnp.sum(o2, axis=0, keepdims=True) * (1.0 / N)
    q = jnp.sum(o2 * o2, axis=0, keepdims=True) * (1.0 / N)
    z = (o2 - m) * lax.rsqrt(q - m * m + BN_EPS) * g2_ref[...] + be2_ref[...]
    out_ref[...] = jax.nn.sigmoid(jnp.maximum(z, 0.0))


def kernel(x, pos_edge_index, edge_attr, W1, b1, bn1_gamma, bn1_beta,
           W2, b2, bn2_gamma, bn2_beta):
    f32 = jnp.float32
    src = pos_edge_index[0].astype(jnp.int32)
    dst = pos_edge_index[1].astype(jnp.int32)
    src3 = src.reshape(16, NCH_R, CH)
    dst3 = dst.reshape(16, NCH_R, CH)
    src_h = src.reshape(32, NCH_S, CH)
    dst_h = dst.reshape(32, NCH_S, CH)
    ones16 = jnp.ones((CH, 16), f32)

    degparts = _sc_degree(dst_h, ones16)
    degcol = (degparts[0] + degparts[1] - 1.0).reshape(NPAD, 1)

    hs2 = pl.pallas_call(
        _tc_prep_body,
        out_shape=jax.ShapeDtypeStruct((2, NPAD, DH), jnp.bfloat16),
    )(x, W1, degcol)

    agg2 = _sc_rowsum(hs2, src3, dst3)

    vcol = pl.pallas_call(
        _tc_mid_body,
        out_shape=jax.ShapeDtypeStruct((NPAD, 1), f32),
    )(agg2, degcol, b1.reshape(1, D), bn1_gamma.reshape(1, D),
      bn1_beta.reshape(1, D), W2.reshape(1, D))
    vs2d = jnp.broadcast_to(vcol, (NPAD, 16))

    parts2 = _sc_segsum16(vs2d, src_h, dst_h)
    psum = (parts2[0] + parts2[1]).reshape(NPAD, 1)

    out = pl.pallas_call(
        _tc_final_body,
        out_shape=jax.ShapeDtypeStruct((N, 1), f32),
    )(psum, degcol, vcol, b2.reshape(1, 1),
      bn2_gamma.reshape(1, 1), bn2_beta.reshape(1, 1))
    return out


# speedup vs baseline: 1.1019x; 1.0198x over previous
import jax
import jax.numpy as jnp
from jax import lax
from jax.experimental import pallas as pl
from jax.experimental.pallas import tpu as pltpu
from jax.experimental.pallas import tpu_sc as plsc

N = 10000
NPAD = 10240
D = 256
DH = 128
E = 160000
BN_EPS = 1e-3
RPT = NPAD // 16
PK = NPAD // 128

CH = 125
NCH_R = E // (16 * CH)
NH_R = NCH_R // 2
NBUF = 5

NCH_S = E // (32 * CH)

_MESH = plsc.VectorSubcoreMesh(core_axis_name="c", subcore_axis_name="s")
_NOTILE = pltpu.CompilerParams(use_tc_tiling_on_sc=False,
                               needs_layout_passes=False)


def _pack_col0(acc, colbuf, pack, out_hbm, cid, sid):
    pltpu.sync_copy(acc.at[pl.ds(sid * RPT, RPT)], colbuf)
    row16 = lax.broadcasted_iota(jnp.int32, (16,), 0)
    zero16 = jnp.zeros((16,), jnp.int32)
    for g in range(RPT // 16):
        vals = plsc.load_gather(colbuf, [g * 16 + row16, zero16])
        pack[g // 8, pl.ds((g % 8) * 16, 16)] = vals
    pltpu.sync_copy(pack, out_hbm.at[cid, pl.ds(sid * (RPT // 128), RPT // 128)])


def _sc_rowsum_body(hs_hbm, src_hbm, dst_hbm, out_hbm,
                    src_idx, dst_idx, rows0, rows1, rows2, rows3, rows4,
                    acc, g0, g1, g2, g3, g4, s0, s1, s2, s3, s4):
    rows = (rows0, rows1, rows2, rows3, rows4)
    gsem = (g0, g1, g2, g3, g4)
    ssem = (s0, s1, s2, s3, s4)
    cid = lax.axis_index("c")
    sid = lax.axis_index("s")
    tab = hs_hbm.at[cid]
    pltpu.sync_copy(tab.at[pl.ds(sid * RPT, RPT)],
                    acc.at[pl.ds(sid * RPT, RPT)])
    plsc.subcore_barrier()
    for h in range(2):
        pltpu.sync_copy(src_hbm.at[sid, pl.ds(h * NH_R, NH_R)], src_idx)
        pltpu.sync_copy(dst_hbm.at[sid, pl.ds(h * NH_R, NH_R)], dst_idx)
        for b in range(NBUF):
            pltpu.async_copy(tab.at[src_idx.at[b]], rows[b], gsem[b])

        def step(i, carry):
            for b in range(NBUF):
                j = NBUF * i + b
                pltpu.make_async_copy(tab.at[src_idx.at[j]], rows[b],
                                      gsem[b]).wait()
                pltpu.async_copy(rows[b], acc.at[dst_idx.at[j]], ssem[b],
                                 add=True)
            for b in range(NBUF):
                j = NBUF * i + b

                @pl.when(j + NBUF < NH_R)
                def _():
                    pltpu.make_async_copy(rows[b], acc.at[dst_idx.at[j]],
                                          ssem[b]).wait()
                    pltpu.async_copy(tab.at[src_idx.at[j + NBUF]], rows[b],
                                     gsem[b])

            return carry

        lax.fori_loop(0, NH_R // NBUF, step, 0)
        for b in range(NBUF):
            pltpu.make_async_copy(rows[b], acc.at[dst_idx.at[0]],
                                  ssem[b]).wait()
    plsc.subcore_barrier()
    pltpu.sync_copy(acc.at[pl.ds(sid * RPT, RPT)],
                    out_hbm.at[cid, pl.ds(sid * RPT, RPT)])


_sc_rowsum = pl.kernel(
    _sc_rowsum_body,
    out_type=jax.ShapeDtypeStruct((2, NPAD, DH), jnp.bfloat16),
    mesh=_MESH,
    compiler_params=_NOTILE,
    scratch_types=[
        pltpu.VMEM((NH_R, CH), jnp.int32),
        pltpu.VMEM((NH_R, CH), jnp.int32),
        pltpu.VMEM((CH, DH), jnp.bfloat16),
        pltpu.VMEM((CH, DH), jnp.bfloat16),
        pltpu.VMEM((CH, DH), jnp.bfloat16),
        pltpu.VMEM((CH, DH), jnp.bfloat16),
        pltpu.VMEM((CH, DH), jnp.bfloat16),
        pltpu.VMEM_SHARED((NPAD, DH), jnp.bfloat16),
    ] + [pltpu.SemaphoreType.DMA] * 10,
)


def _sc_segsum16_body(tab_hbm, src_hbm, dst_hbm, out_hbm,
                      src_idx, dst_idx, rows0, rows1, colbuf, pack, acc,
                      sem0, sem1):
    cid = lax.axis_index("c")
    sid = lax.axis_index("s")
    wid = cid * 16 + sid
    pltpu.sync_copy(src_hbm.at[wid], src_idx)
    pltpu.sync_copy(dst_hbm.at[wid], dst_idx)
    pltpu.sync_copy(tab_hbm.at[pl.ds(sid * RPT, RPT)],
                    acc.at[pl.ds(sid * RPT, RPT)])
    plsc.subcore_barrier()
    pltpu.async_copy(tab_hbm.at[src_idx.at[0]], rows0, sem0)
    pltpu.async_copy(tab_hbm.at[src_idx.at[1]], rows1, sem1)

    def step(i, carry):
        j0 = 2 * i
        j1 = 2 * i + 1
        pltpu.make_async_copy(tab_hbm.at[src_idx.at[j0]], rows0, sem0).wait()
        pltpu.sync_copy(rows0, acc.at[dst_idx.at[j0]], add=True)

        @pl.when(j0 + 2 < NCH_S)
        def _():
            pltpu.async_copy(tab_hbm.at[src_idx.at[j0 + 2]], rows0, sem0)

        pltpu.make_async_copy(tab_hbm.at[src_idx.at[j1]], rows1, sem1).wait()
        pltpu.sync_copy(rows1, acc.at[dst_idx.at[j1]], add=True)

        @pl.when(j1 + 2 < NCH_S)
        def _():
            pltpu.async_copy(tab_hbm.at[src_idx.at[j1 + 2]], rows1, sem1)

        return carry

    lax.fori_loop(0, NCH_S // 2, step, 0)
    plsc.subcore_barrier()
    _pack_col0(acc, colbuf, pack, out_hbm, cid, sid)


_sc_segsum16 = pl.kernel(
    _sc_segsum16_body,
    out_type=jax.ShapeDtypeStruct((2, PK, 128), jnp.float32),
    mesh=_MESH,
    compiler_params=_NOTILE,
    scratch_types=[
        pltpu.VMEM((NCH_S, CH), jnp.int32),
        pltpu.VMEM((NCH_S, CH), jnp.int32),
        pltpu.VMEM((CH, 16), jnp.float32),
        pltpu.VMEM((CH, 16), jnp.float32),
        pltpu.VMEM((RPT, 16), jnp.float32),
        pltpu.VMEM((RPT // 128, 128), jnp.float32),
        pltpu.VMEM_SHARED((NPAD, 16), jnp.float32),
        pltpu.SemaphoreType.DMA,
        pltpu.SemaphoreType.DMA,
    ],
)


def _sc_degree_body(dst_hbm, ones_hbm, out_hbm, dst_idx, ones_v, colbuf,
                    pack, acc):
    cid = lax.axis_index("c")
    sid = lax.axis_index("s")
    wid = cid * 16 + sid
    pltpu.sync_copy(dst_hbm.at[wid], dst_idx)
    pltpu.sync_copy(ones_hbm, ones_v)
    for k in range(RPT // CH + 1):
        base = sid * RPT + k * CH
        size = min(CH, RPT - k * CH)
        if size > 0:
            pltpu.sync_copy(ones_v.at[pl.ds(0, size)],
                            acc.at[pl.ds(base, size)])
    plsc.subcore_barrier()

    def step(j, carry):
        pltpu.sync_copy(ones_v, acc.at[dst_idx.at[j]], add=True)
        return carry

    lax.fori_loop(0, NCH_S, step, 0)
    plsc.subcore_barrier()
    _pack_col0(acc, colbuf, pack, out_hbm, cid, sid)


_sc_degree = pl.kernel(
    _sc_degree_body,
    out_type=jax.ShapeDtypeStruct((2, PK, 128), jnp.float32),
    mesh=_MESH,
    compiler_params=_NOTILE,
    scratch_types=[
        pltpu.VMEM((NCH_S, CH), jnp.int32),
        pltpu.VMEM((CH, 16), jnp.float32),
        pltpu.VMEM((RPT, 16), jnp.float32),
        pltpu.VMEM((RPT // 128, 128), jnp.float32),
        pltpu.VMEM_SHARED((NPAD, 16), jnp.float32),
    ],
)


def _tc_matmul_body(x_ref, w1_ref, h_ref):
    h_ref[...] = jnp.dot(x_ref[...], w1_ref[...],
                         preferred_element_type=jnp.float32)


def _tc_mid_body(agg_ref, dc_ref, b1_ref, g1_ref, be1_ref, w2_ref, out_ref):
    dinv = lax.rsqrt(dc_ref[...])
    aggl = agg_ref[0].astype(jnp.float32)
    aggr = agg_ref[1].astype(jnp.float32)
    zl = jnp.maximum(aggl * dinv + b1_ref[:, 0:DH], 0.0)
    zr = jnp.maximum(aggr * dinv + b1_ref[:, DH:D], 0.0)
    ml = jnp.sum(zl[0:N, :], axis=0, keepdims=True) * (1.0 / N)
    mr = jnp.sum(zr[0:N, :], axis=0, keepdims=True) * (1.0 / N)
    ql = jnp.sum(zl[0:N, :] * zl[0:N, :], axis=0, keepdims=True) * (1.0 / N)
    qr = jnp.sum(zr[0:N, :] * zr[0:N, :], axis=0, keepdims=True) * (1.0 / N)
    il = lax.rsqrt(ql - ml * ml + BN_EPS)
    ir = lax.rsqrt(qr - mr * mr + BN_EPS)
    hl = jnp.maximum((zl - ml) * il * g1_ref[:, 0:DH] + be1_ref[:, 0:DH], 0.0)
    hr = jnp.maximum((zr - mr) * ir * g1_ref[:, DH:D] + be1_ref[:, DH:D], 0.0)
    v = (jnp.sum(hl * w2_ref[:, 0:DH], axis=1, keepdims=True)
         + jnp.sum(hr * w2_ref[:, DH:D], axis=1, keepdims=True))
    out_ref[...] = v * dinv


def _tc_final_body(p_ref, dc_ref, vs_ref, b2_ref, g2_ref, be2_ref, out_ref):
    a2 = p_ref[0:N, :] - vs_ref[0:N, :]
    o2 = a2 * lax.rsqrt(dc_ref[0:N, :]) + b2_ref[...]
    m = jnp.sum(o2, axis=0, keepdims=True) * (1.0 / N)
    q = jnp.sum(o2 * o2, axis=0, keepdims=True) * (1.0 / N)
    z = (o2 - m) * lax.rsqrt(q - m * m + BN_EPS) * g2_ref[...] + be2_ref[...]
    out_ref[...] = jax.nn.sigmoid(jnp.maximum(z, 0.0))


def kernel(x, pos_edge_index, edge_attr, W1, b1, bn1_gamma, bn1_beta,
           W2, b2, bn2_gamma, bn2_beta):
    f32 = jnp.float32
    src = pos_edge_index[0].astype(jnp.int32)
    dst = pos_edge_index[1].astype(jnp.int32)
    src3 = src.reshape(16, NCH_R, CH)
    dst3 = dst.reshape(16, NCH_R, CH)
    src_h = src.reshape(32, NCH_S, CH)
    dst_h = dst.reshape(32, NCH_S, CH)
    ones16 = jnp.ones((CH, 16), f32)

    degparts = _sc_degree(dst_h, ones16)
    degcol = (degparts[0] + degparts[1] - 1.0).reshape(NPAD, 1)

    h = pl.pallas_call(
        _tc_matmul_body,
        out_shape=jax.ShapeDtypeStruct((N, D), f32),
    )(x, W1)
    hs = (h * lax.rsqrt(degcol[0:N, :])).astype(jnp.bfloat16)
    hs = jnp.pad(hs, ((0, NPAD - N), (0, 0)))
    hs2 = jnp.stack([hs[:, 0:DH], hs[:, DH:D]])

    agg2 = _sc_rowsum(hs2, src3, dst3)

    vcol = pl.pallas_call(
        _tc_mid_body,
        out_shape=jax.ShapeDtypeStruct((NPAD, 1), f32),
    )(agg2, degcol, b1.reshape(1, D), bn1_gamma.reshape(1, D),
      bn1_beta.reshape(1, D), W2.reshape(1, D))
    vs2d = jnp.broadcast_to(vcol, (NPAD, 16))

    parts2 = _sc_segsum16(vs2d, src_h, dst_h)
    psum = (parts2[0] + parts2[1]).reshape(NPAD, 1)

    out = pl.pallas_call(
        _tc_final_body,
        out_shape=jax.ShapeDtypeStruct((N, 1), f32),
    )(psum, degcol, vcol, b2.reshape(1, 1),
      bn2_gamma.reshape(1, 1), bn2_beta.reshape(1, 1))
    return out


# speedup vs baseline: 1.1523x; 1.0458x over previous
import jax
import jax.numpy as jnp
from jax import lax
from jax.experimental import pallas as pl
from jax.experimental.pallas import tpu as pltpu
from jax.experimental.pallas import tpu_sc as plsc

N = 10000
NPAD = 10240
D = 256
DH = 128
E = 160000
BN_EPS = 1e-3
RPT = NPAD // 16
PK = NPAD // 128

CH = 125
NCH_R = E // (16 * CH)
NH_R = NCH_R // 2
NBUF = 8

NCH_S = E // (32 * CH)

_MESH = plsc.VectorSubcoreMesh(core_axis_name="c", subcore_axis_name="s")
_NOTILE = pltpu.CompilerParams(use_tc_tiling_on_sc=False,
                               needs_layout_passes=False)


def _pack_col0(acc, colbuf, pack, out_hbm, cid, sid):
    pltpu.sync_copy(acc.at[pl.ds(sid * RPT, RPT)], colbuf)
    row16 = lax.broadcasted_iota(jnp.int32, (16,), 0)
    zero16 = jnp.zeros((16,), jnp.int32)
    for g in range(RPT // 16):
        vals = plsc.load_gather(colbuf, [g * 16 + row16, zero16])
        pack[g // 8, pl.ds((g % 8) * 16, 16)] = vals
    pltpu.sync_copy(pack, out_hbm.at[cid, pl.ds(sid * (RPT // 128), RPT // 128)])


def _sc_rowsum_body(hs_hbm, src_hbm, dst_hbm, out_hbm,
                    src_idx, dst_idx, rows0, rows1, rows2, rows3, rows4,
                    rows5, rows6, rows7, acc, g0, g1, g2, g3, g4, g5, g6,
                    g7, s0, s1, s2, s3, s4, s5, s6, s7):
    rows = (rows0, rows1, rows2, rows3, rows4, rows5, rows6, rows7)
    gsem = (g0, g1, g2, g3, g4, g5, g6, g7)
    ssem = (s0, s1, s2, s3, s4, s5, s6, s7)
    cid = lax.axis_index("c")
    sid = lax.axis_index("s")
    tab = hs_hbm.at[cid]
    pltpu.sync_copy(tab.at[pl.ds(sid * RPT, RPT)],
                    acc.at[pl.ds(sid * RPT, RPT)])
    plsc.subcore_barrier()
    for h in range(2):
        pltpu.sync_copy(src_hbm.at[sid, pl.ds(h * NH_R, NH_R)], src_idx)
        pltpu.sync_copy(dst_hbm.at[sid, pl.ds(h * NH_R, NH_R)], dst_idx)
        for b in range(NBUF):
            pltpu.async_copy(tab.at[src_idx.at[b]], rows[b], gsem[b])

        def step(i, carry):
            for b in range(NBUF):
                j = NBUF * i + b
                pltpu.make_async_copy(tab.at[src_idx.at[j]], rows[b],
                                      gsem[b]).wait()
                pltpu.async_copy(rows[b], acc.at[dst_idx.at[j]], ssem[b],
                                 add=True)
            for b in range(NBUF):
                j = NBUF * i + b

                @pl.when(j + NBUF < NH_R)
                def _():
                    pltpu.make_async_copy(rows[b], acc.at[dst_idx.at[j]],
                                          ssem[b]).wait()
                    pltpu.async_copy(tab.at[src_idx.at[j + NBUF]], rows[b],
                                     gsem[b])

            return carry

        lax.fori_loop(0, NH_R // NBUF, step, 0)
        for b in range(NBUF):
            pltpu.make_async_copy(rows[b], acc.at[dst_idx.at[0]],
                                  ssem[b]).wait()
    plsc.subcore_barrier()
    pltpu.sync_copy(acc.at[pl.ds(sid * RPT, RPT)],
                    out_hbm.at[cid, pl.ds(sid * RPT, RPT)])


_sc_rowsum = pl.kernel(
    _sc_rowsum_body,
    out_type=jax.ShapeDtypeStruct((2, NPAD, DH), jnp.bfloat16),
    mesh=_MESH,
    compiler_params=_NOTILE,
    scratch_types=[
        pltpu.VMEM((NH_R, CH), jnp.int32),
        pltpu.VMEM((NH_R, CH), jnp.int32),
        pltpu.VMEM((CH, DH), jnp.bfloat16),
        pltpu.VMEM((CH, DH), jnp.bfloat16),
        pltpu.VMEM((CH, DH), jnp.bfloat16),
        pltpu.VMEM((CH, DH), jnp.bfloat16),
        pltpu.VMEM((CH, DH), jnp.bfloat16),
        pltpu.VMEM((CH, DH), jnp.bfloat16),
        pltpu.VMEM((CH, DH), jnp.bfloat16),
        pltpu.VMEM((CH, DH), jnp.bfloat16),
        pltpu.VMEM_SHARED((NPAD, DH), jnp.bfloat16),
    ] + [pltpu.SemaphoreType.DMA] * 16,
)


def _sc_segsum16_body(tab_hbm, src_hbm, dst_hbm, out_hbm,
                      src_idx, dst_idx, rows0, rows1, rows2, rows3, colbuf,
                      pack, acc, sem0, sem1, sem2, sem3):
    cid = lax.axis_index("c")
    sid = lax.axis_index("s")
    wid = cid * 16 + sid
    pltpu.sync_copy(src_hbm.at[wid], src_idx)
    pltpu.sync_copy(dst_hbm.at[wid], dst_idx)
    pltpu.sync_copy(tab_hbm.at[pl.ds(sid * RPT, RPT)],
                    acc.at[pl.ds(sid * RPT, RPT)])
    plsc.subcore_barrier()
    srows = (rows0, rows1, rows2, rows3)
    ssems = (sem0, sem1, sem2, sem3)
    for b in range(4):
        pltpu.async_copy(tab_hbm.at[src_idx.at[b]], srows[b], ssems[b])

    def step(i, carry):
        for b in range(4):
            j = 4 * i + b
            pltpu.make_async_copy(tab_hbm.at[src_idx.at[j]], srows[b],
                                  ssems[b]).wait()
            pltpu.sync_copy(srows[b], acc.at[dst_idx.at[j]], add=True)

            @pl.when(j + 4 < NCH_S)
            def _():
                pltpu.async_copy(tab_hbm.at[src_idx.at[j + 4]], srows[b],
                                 ssems[b])

        return carry

    lax.fori_loop(0, NCH_S // 4, step, 0)
    plsc.subcore_barrier()
    _pack_col0(acc, colbuf, pack, out_hbm, cid, sid)


_sc_segsum16 = pl.kernel(
    _sc_segsum16_body,
    out_type=jax.ShapeDtypeStruct((2, PK, 128), jnp.float32),
    mesh=_MESH,
    compiler_params=_NOTILE,
    scratch_types=[
        pltpu.VMEM((NCH_S, CH), jnp.int32),
        pltpu.VMEM((NCH_S, CH), jnp.int32),
        pltpu.VMEM((CH, 16), jnp.float32),
        pltpu.VMEM((CH, 16), jnp.float32),
        pltpu.VMEM((CH, 16), jnp.float32),
        pltpu.VMEM((CH, 16), jnp.float32),
        pltpu.VMEM((RPT, 16), jnp.float32),
        pltpu.VMEM((RPT // 128, 128), jnp.float32),
        pltpu.VMEM_SHARED((NPAD, 16), jnp.float32),
    ] + [pltpu.SemaphoreType.DMA] * 4,
)


def _sc_degree_body(dst_hbm, ones_hbm, out_hbm, dst_idx, ones_v, colbuf,
                    pack, acc):
    cid = lax.axis_index("c")
    sid = lax.axis_index("s")
    wid = cid * 16 + sid
    pltpu.sync_copy(dst_hbm.at[wid], dst_idx)
    pltpu.sync_copy(ones_hbm, ones_v)
    for k in range(RPT // CH + 1):
        base = sid * RPT + k * CH
        size = min(CH, RPT - k * CH)
        if size > 0:
            pltpu.sync_copy(ones_v.at[pl.ds(0, size)],
                            acc.at[pl.ds(base, size)])
    plsc.subcore_barrier()

    def step(j, carry):
        pltpu.sync_copy(ones_v, acc.at[dst_idx.at[j]], add=True)
        return carry

    lax.fori_loop(0, NCH_S, step, 0)
    plsc.subcore_barrier()
    _pack_col0(acc, colbuf, pack, out_hbm, cid, sid)


_sc_degree = pl.kernel(
    _sc_degree_body,
    out_type=jax.ShapeDtypeStruct((2, PK, 128), jnp.float32),
    mesh=_MESH,
    compiler_params=_NOTILE,
    scratch_types=[
        pltpu.VMEM((NCH_S, CH), jnp.int32),
        pltpu.VMEM((CH, 16), jnp.float32),
        pltpu.VMEM((RPT, 16), jnp.float32),
        pltpu.VMEM((RPT // 128, 128), jnp.float32),
        pltpu.VMEM_SHARED((NPAD, 16), jnp.float32),
    ],
)


def _tc_matmul_body(x_ref, w1_ref, h_ref):
    h_ref[...] = jnp.dot(x_ref[...], w1_ref[...],
                         preferred_element_type=jnp.float32)


def _tc_mid_body(agg_ref, dc_ref, b1_ref, g1_ref, be1_ref, w2_ref, out_ref):
    dinv = lax.rsqrt(dc_ref[...])
    aggl = agg_ref[0].astype(jnp.float32)
    aggr = agg_ref[1].astype(jnp.float32)
    zl = jnp.maximum(aggl * dinv + b1_ref[:, 0:DH], 0.0)
    zr = jnp.maximum(aggr * dinv + b1_ref[:, DH:D], 0.0)
    ml = jnp.sum(zl[0:N, :], axis=0, keepdims=True) * (1.0 / N)
    mr = jnp.sum(zr[0:N, :], axis=0, keepdims=True) * (1.0 / N)
    ql = jnp.sum(zl[0:N, :] * zl[0:N, :], axis=0, keepdims=True) * (1.0 / N)
    qr = jnp.sum(zr[0:N, :] * zr[0:N, :], axis=0, keepdims=True) * (1.0 / N)
    il = lax.rsqrt(ql - ml * ml + BN_EPS)
    ir = lax.rsqrt(qr - mr * mr + BN_EPS)
    hl = jnp.maximum((zl - ml) * il * g1_ref[:, 0:DH] + be1_ref[:, 0:DH], 0.0)
    hr = jnp.maximum((zr - mr) * ir * g1_ref[:, DH:D] + be1_ref[:, DH:D], 0.0)
    v = (jnp.sum(hl * w2_ref[:, 0:DH], axis=1, keepdims=True)
         + jnp.sum(hr * w2_ref[:, DH:D], axis=1, keepdims=True))
    out_ref[...] = v * dinv


def _tc_final_body(p_ref, dc_ref, vs_ref, b2_ref, g2_ref, be2_ref, out_ref):
    a2 = p_ref[0:N, :] - vs_ref[0:N, :]
    o2 = a2 * lax.rsqrt(dc_ref[0:N, :]) + b2_ref[...]
    m = jnp.sum(o2, axis=0, keepdims=True) * (1.0 / N)
    q = jnp.sum(o2 * o2, axis=0, keepdims=True) * (1.0 / N)
    z = (o2 - m) * lax.rsqrt(q - m * m + BN_EPS) * g2_ref[...] + be2_ref[...]
    out_ref[...] = jax.nn.sigmoid(jnp.maximum(z, 0.0))


def kernel(x, pos_edge_index, edge_attr, W1, b1, bn1_gamma, bn1_beta,
           W2, b2, bn2_gamma, bn2_beta):
    f32 = jnp.float32
    src = pos_edge_index[0].astype(jnp.int32)
    dst = pos_edge_index[1].astype(jnp.int32)
    src3 = src.reshape(16, NCH_R, CH)
    dst3 = dst.reshape(16, NCH_R, CH)
    src_h = src.reshape(32, NCH_S, CH)
    dst_h = dst.reshape(32, NCH_S, CH)
    ones16 = jnp.ones((CH, 16), f32)

    degparts = _sc_degree(dst_h, ones16)
    degcol = (degparts[0] + degparts[1] - 1.0).reshape(NPAD, 1)

    h = pl.pallas_call(
        _tc_matmul_body,
        out_shape=jax.ShapeDtypeStruct((N, D), f32),
    )(x, W1)
    hs = (h * lax.rsqrt(degcol[0:N, :])).astype(jnp.bfloat16)
    hs = jnp.pad(hs, ((0, NPAD - N), (0, 0)))
    hs2 = jnp.stack([hs[:, 0:DH], hs[:, DH:D]])

    agg2 = _sc_rowsum(hs2, src3, dst3)

    vcol = pl.pallas_call(
        _tc_mid_body,
        out_shape=jax.ShapeDtypeStruct((NPAD, 1), f32),
    )(agg2, degcol, b1.reshape(1, D), bn1_gamma.reshape(1, D),
      bn1_beta.reshape(1, D), W2.reshape(1, D))
    vs2d = jnp.broadcast_to(vcol, (NPAD, 16))

    parts2 = _sc_segsum16(vs2d, src_h, dst_h)
    psum = (parts2[0] + parts2[1]).reshape(NPAD, 1)

    out = pl.pallas_call(
        _tc_final_body,
        out_shape=jax.ShapeDtypeStruct((N, 1), f32),
    )(psum, degcol, vcol, b2.reshape(1, 1),
      bn2_gamma.reshape(1, 1), bn2_beta.reshape(1, 1))
    return out
